# Initial kernel scaffold; baseline (speedup 1.0000x reference)
import jax
import jax.numpy as jnp
from jax.experimental import pallas as pl


def kernel(initial_node_representation, adjacency_list_0, adjacency_list_1, W_msg_0_0, b_msg_0_0, W_msg_0_1, b_msg_0_1, W_ih_0, W_hh_0, b_ih_0, b_hh_0, fc_W, fc_b):
    raise NotImplementedError("write your pallas kernel here")


# speedup vs baseline: 3.7292x; 3.7292x over previous
import functools

import jax
import jax.numpy as jnp
from jax import lax
from jax.experimental import pallas as pl
from jax.experimental.pallas import tpu as pltpu
from jax.experimental.pallas import tpu_sc as plsc

_N = 10000
_H = 128
_E = 160000

_NC = 2
_NS = 16
_CHUNK = 128
_CPT = 80
_EPAD = _NS * _CPT * _CHUNK
_NPAD_A = 10240
_RPT_Z = _NPAD_A // _NS
_NP1 = 10240
_IPT = _NP1 // _NS
_NCW = 10
_RPT_O = _N // _NCW
_RB = 1000


@functools.partial(
    pl.kernel,
    out_type=(
        jax.ShapeDtypeStruct((_NC, _N, _H), jnp.float32),
        jax.ShapeDtypeStruct((_NC, _NP1), jnp.float32),
    ),
    mesh=plsc.VectorSubcoreMesh(core_axis_name="c", subcore_axis_name="s"),
    scratch_types=(
        pltpu.VMEM_SHARED((_NPAD_A, _H), jnp.float32),
        pltpu.VMEM_SHARED((_NP1,), jnp.float32),
        pltpu.VMEM((_CPT, _CHUNK), jnp.int32),
        pltpu.VMEM((_CPT, _CHUNK), jnp.int32),
        pltpu.VMEM((_CHUNK, _H), jnp.float32),
        pltpu.VMEM((_CHUNK,), jnp.float32),
        pltpu.SemaphoreType.DMA,
    ),
)
def _sc_agg(h_hbm, src_hbm, dst_hbm, zrows_hbm, zdeg_hbm, ones_hbm,
            a_out, deg_out,
            a_sh, deg_sh, src_v, dst_v, rows_v, ones_v, sem):
    c = lax.axis_index("c")
    s = lax.axis_index("s")

    pltpu.sync_copy(zrows_hbm, rows_v)
    pltpu.sync_copy(ones_hbm, ones_v)
    pltpu.sync_copy(src_hbm.at[c, pl.ds(s * _CPT, _CPT)], src_v)
    pltpu.sync_copy(dst_hbm.at[c, pl.ds(s * _CPT, _CPT)], dst_v)

    base_z = s * _RPT_Z

    def _za(i, carry):
        pltpu.sync_copy(rows_v, a_sh.at[pl.ds(base_z + i * _CHUNK, _CHUNK)])
        return carry

    lax.fori_loop(0, _RPT_Z // _CHUNK, _za, 0)
    pltpu.sync_copy(zdeg_hbm.at[pl.ds(s * _IPT, _IPT)],
                    deg_sh.at[pl.ds(s * _IPT, _IPT)])
    plsc.subcore_barrier()

    def _step(j, carry):
        pltpu.async_copy(h_hbm.at[src_v.at[j]], rows_v, sem).wait()
        pltpu.sync_copy(rows_v, a_sh.at[dst_v.at[j]], add=True)
        pltpu.sync_copy(ones_v, deg_sh.at[dst_v.at[j]], add=True)
        return carry

    lax.fori_loop(0, _CPT, _step, 0)
    plsc.subcore_barrier()

    @pl.when(s < _NCW)
    def _():
        rb = s * _RPT_O
        pltpu.sync_copy(a_sh.at[pl.ds(rb, _RPT_O)],
                        a_out.at[c, pl.ds(rb, _RPT_O)])

    pltpu.sync_copy(deg_sh.at[pl.ds(s * _IPT, _IPT)],
                    deg_out.at[c, pl.ds(s * _IPT, _IPT)])


def _tc_body(final, a_ref, h_ref, deg_ref, wm_ref, bm_ref, wih_ref, whh_ref,
             bih_ref, bhh_ref, fcw_ref, fcb_ref, ho_ref, go_ref=None):
    i = pl.program_id(0)
    acat = jnp.concatenate([a_ref[0], a_ref[1]], axis=1)
    inc = jnp.dot(acat, wm_ref[...], preferred_element_type=jnp.float32)
    inc = inc + jnp.dot(deg_ref[...], bm_ref[...],
                        preferred_element_type=jnp.float32)
    h = h_ref[...]
    gi = jnp.dot(inc, wih_ref[...], preferred_element_type=jnp.float32) + bih_ref[...]
    gh = jnp.dot(h, whh_ref[...], preferred_element_type=jnp.float32) + bhh_ref[...]
    r = jax.nn.sigmoid(gi[:, :_H] + gh[:, :_H])
    z = jax.nn.sigmoid(gi[:, _H:2 * _H] + gh[:, _H:2 * _H])
    n = jnp.tanh(gi[:, 2 * _H:] + r * gh[:, 2 * _H:])
    hn = (1.0 - z) * n + z * h
    ho_ref[...] = hn
    if final:
        o = jnp.dot(hn, fcw_ref[...], preferred_element_type=jnp.float32) + fcb_ref[...]
        m = jnp.max(o, axis=0, keepdims=True)

        @pl.when(i == 0)
        def _():
            go_ref[...] = m

        @pl.when(i > 0)
        def _():
            go_ref[...] = jnp.maximum(go_ref[...], m)


def _make_tc_step(final):
    in_specs = [
        pl.BlockSpec((_NC, _RB, _H), lambda i: (0, i, 0)),
        pl.BlockSpec((_RB, _H), lambda i: (i, 0)),
        pl.BlockSpec((_RB, 8), lambda i: (i, 0)),
        pl.BlockSpec((2 * _H, _H), lambda i: (0, 0)),
        pl.BlockSpec((8, _H), lambda i: (0, 0)),
        pl.BlockSpec((_H, 3 * _H), lambda i: (0, 0)),
        pl.BlockSpec((_H, 3 * _H), lambda i: (0, 0)),
        pl.BlockSpec((1, 3 * _H), lambda i: (0, 0)),
        pl.BlockSpec((1, 3 * _H), lambda i: (0, 0)),
        pl.BlockSpec((_H, _H), lambda i: (0, 0)),
        pl.BlockSpec((1, _H), lambda i: (0, 0)),
    ]
    out_specs = pl.BlockSpec((_RB, _H), lambda i: (i, 0))
    out_shape = jax.ShapeDtypeStruct((_N, _H), jnp.float32)
    if final:
        out_specs = [out_specs, pl.BlockSpec((1, _H), lambda i: (0, 0))]
        out_shape = [out_shape, jax.ShapeDtypeStruct((1, _H), jnp.float32)]
    return pl.pallas_call(
        functools.partial(_tc_body, final),
        grid=(_N // _RB,),
        in_specs=in_specs,
        out_specs=out_specs,
        out_shape=out_shape,
    )


_tc_step_mid = _make_tc_step(False)
_tc_step_fin = _make_tc_step(True)


def kernel(initial_node_representation, adjacency_list_0, adjacency_list_1,
           W_msg_0_0, b_msg_0_0, W_msg_0_1, b_msg_0_1,
           W_ih_0, W_hh_0, b_ih_0, b_hh_0, fc_W, fc_b):
    def prep(adj):
        src = adj[:, 0].astype(jnp.int32)
        dst = adj[:, 1].astype(jnp.int32)
        src = jnp.concatenate([src, jnp.zeros((_EPAD - _E,), jnp.int32)])
        dst = jnp.concatenate([dst, jnp.full((_EPAD - _E,), _N, jnp.int32)])
        return src.reshape(_NS * _CPT, _CHUNK), dst.reshape(_NS * _CPT, _CHUNK)

    s0, d0 = prep(adjacency_list_0)
    s1, d1 = prep(adjacency_list_1)
    src = jnp.stack([s0, s1])
    dst = jnp.stack([d0, d1])

    zrows = jnp.zeros((_CHUNK, _H), jnp.float32)
    zdeg = jnp.zeros((_NP1,), jnp.float32)
    ones = jnp.ones((_CHUNK,), jnp.float32)

    wm = jnp.concatenate([W_msg_0_0.T, W_msg_0_1.T], axis=0)
    bm = jnp.zeros((8, _H), jnp.float32).at[0].set(b_msg_0_0).at[1].set(b_msg_0_1)
    wih = W_ih_0.T
    whh = W_hh_0.T
    bih = b_ih_0[None, :]
    bhh = b_hh_0[None, :]
    fcw = fc_W.T
    fcb = fc_b[None, :]

    h = initial_node_representation
    g = None
    for t in range(2):
        a, deg = _sc_agg(h, src, dst, zrows, zdeg, ones)
        deg8 = jnp.concatenate(
            [deg[0, :_N, None], deg[1, :_N, None],
             jnp.zeros((_N, 6), jnp.float32)], axis=1)
        if t == 0:
            h = _tc_step_mid(a, h, deg8, wm, bm, wih, whh, bih, bhh, fcw, fcb)
        else:
            h, g = _tc_step_fin(a, h, deg8, wm, bm, wih, whh, bih, bhh, fcw, fcb)
    return g[0]


# speedup vs baseline: 3.9923x; 1.0706x over previous
import functools

import jax
import jax.numpy as jnp
from jax import lax
from jax.experimental import pallas as pl
from jax.experimental.pallas import tpu as pltpu
from jax.experimental.pallas import tpu_sc as plsc

_N = 10000
_H = 128
_E = 160000

_NC = 2
_NS = 16
_CHUNK = 128
_CPT = 80
_EPAD = _NS * _CPT * _CHUNK
_NPAD_A = 10240
_RPT_Z = _NPAD_A // _NS
_NP1 = 10240
_IPT = _NP1 // _NS
_NCW = 10
_RPT_O = _N // _NCW
_RB = 1000


def _make_sc_agg(with_deg):
    out_type = [jax.ShapeDtypeStruct((_NC, _N, _H), jnp.float32)]
    scratch = [
        pltpu.VMEM_SHARED((_NPAD_A, _H), jnp.float32),
        pltpu.VMEM_SHARED((_NP1,), jnp.float32),
        pltpu.VMEM((_CPT // 2, _CHUNK), jnp.int32),
        pltpu.VMEM((_CPT // 2, _CHUNK), jnp.int32),
        pltpu.VMEM((2, _CHUNK, _H), jnp.float32),
        pltpu.VMEM((_CHUNK,), jnp.float32),
        pltpu.SemaphoreType.DMA,
        pltpu.SemaphoreType.DMA,
        pltpu.SemaphoreType.DMA,
    ]
    if with_deg:
        out_type.append(jax.ShapeDtypeStruct((_NC, _NP1), jnp.float32))

    def body(h_hbm, src_hbm, dst_hbm, zrows_hbm, zdeg_hbm, ones_hbm,
             *rest):
        if with_deg:
            (a_out, deg_out, a_sh, deg_sh, src_v, dst_v, rows_v, ones_v,
             gsem, ssem, dsem) = rest
        else:
            (a_out, a_sh, deg_sh, src_v, dst_v, rows_v, ones_v,
             gsem, ssem, dsem) = rest
        c = lax.axis_index("c")
        s = lax.axis_index("s")

        pltpu.sync_copy(zrows_hbm, rows_v.at[0])
        if with_deg:
            pltpu.sync_copy(ones_hbm, ones_v)

        base_z = s * _RPT_Z

        def _za(i, carry):
            pltpu.sync_copy(rows_v.at[0],
                            a_sh.at[pl.ds(base_z + i * _CHUNK, _CHUNK)])
            return carry

        lax.fori_loop(0, _RPT_Z // _CHUNK, _za, 0)
        if with_deg:
            pltpu.sync_copy(zdeg_hbm.at[pl.ds(s * _IPT, _IPT)],
                            deg_sh.at[pl.ds(s * _IPT, _IPT)])
        plsc.subcore_barrier()

        hcpt = _CPT // 2
        for half in range(2):
            pltpu.sync_copy(
                src_hbm.at[c, pl.ds(s * _CPT + half * hcpt, hcpt)], src_v)
            pltpu.sync_copy(
                dst_hbm.at[c, pl.ds(s * _CPT + half * hcpt, hcpt)], dst_v)
            pltpu.async_copy(h_hbm.at[src_v.at[0]], rows_v.at[0], gsem)

            def _step(j, carry):
                r = lax.rem(j, 2)
                pltpu.make_async_copy(h_hbm.at[pl.ds(0, _CHUNK)],
                                      rows_v.at[r], gsem).wait()
                pltpu.async_copy(rows_v.at[r], a_sh.at[dst_v.at[j]],
                                 ssem, add=True)
                if with_deg:
                    pltpu.async_copy(ones_v, deg_sh.at[dst_v.at[j]],
                                     dsem, add=True)

                @pl.when(j + 1 < hcpt)
                def _():
                    pltpu.async_copy(h_hbm.at[src_v.at[j + 1]],
                                     rows_v.at[1 - r], gsem)

                pltpu.make_async_copy(h_hbm.at[pl.ds(0, _CHUNK)],
                                      a_sh.at[pl.ds(0, _CHUNK)], ssem).wait()
                if with_deg:
                    pltpu.make_async_copy(zdeg_hbm.at[pl.ds(0, _CHUNK)],
                                          deg_sh.at[pl.ds(0, _CHUNK)],
                                          dsem).wait()
                return carry

            lax.fori_loop(0, hcpt, _step, 0)
        plsc.subcore_barrier()

        @pl.when(s < _NCW)
        def _():
            rb = s * _RPT_O
            pltpu.sync_copy(a_sh.at[pl.ds(rb, _RPT_O)],
                            a_out.at[c, pl.ds(rb, _RPT_O)])

        if with_deg:
            pltpu.sync_copy(deg_sh.at[pl.ds(s * _IPT, _IPT)],
                            deg_out.at[c, pl.ds(s * _IPT, _IPT)])

    return pl.kernel(
        body,
        out_type=tuple(out_type) if with_deg else out_type[0],
        mesh=plsc.VectorSubcoreMesh(core_axis_name="c", subcore_axis_name="s"),
        scratch_types=tuple(scratch),
    )


_sc_agg = _make_sc_agg(True)


def _tc_body(final, a_ref, h_ref, deg_ref, wm_ref, bm_ref, wih_ref, whh_ref,
             bih_ref, bhh_ref, fcw_ref, fcb_ref, ho_ref, go_ref=None):
    i = pl.program_id(0)
    acat = jnp.concatenate([a_ref[0], a_ref[1]], axis=1)
    inc = jnp.dot(acat, wm_ref[...], preferred_element_type=jnp.float32)
    inc = inc + jnp.dot(deg_ref[...], bm_ref[...],
                        preferred_element_type=jnp.float32)
    h = h_ref[...]
    gi = jnp.dot(inc, wih_ref[...], preferred_element_type=jnp.float32) + bih_ref[...]
    gh = jnp.dot(h, whh_ref[...], preferred_element_type=jnp.float32) + bhh_ref[...]
    r = jax.nn.sigmoid(gi[:, :_H] + gh[:, :_H])
    z = jax.nn.sigmoid(gi[:, _H:2 * _H] + gh[:, _H:2 * _H])
    n = jnp.tanh(gi[:, 2 * _H:] + r * gh[:, 2 * _H:])
    hn = (1.0 - z) * n + z * h
    ho_ref[...] = hn
    if final:
        o = jnp.dot(hn, fcw_ref[...], preferred_element_type=jnp.float32) + fcb_ref[...]
        m = jnp.max(o, axis=0, keepdims=True)

        @pl.when(i == 0)
        def _():
            go_ref[...] = m

        @pl.when(i > 0)
        def _():
            go_ref[...] = jnp.maximum(go_ref[...], m)


def _make_tc_step(final):
    in_specs = [
        pl.BlockSpec((_NC, _RB, _H), lambda i: (0, i, 0)),
        pl.BlockSpec((_RB, _H), lambda i: (i, 0)),
        pl.BlockSpec((_RB, 8), lambda i: (i, 0)),
        pl.BlockSpec((2 * _H, _H), lambda i: (0, 0)),
        pl.BlockSpec((8, _H), lambda i: (0, 0)),
        pl.BlockSpec((_H, 3 * _H), lambda i: (0, 0)),
        pl.BlockSpec((_H, 3 * _H), lambda i: (0, 0)),
        pl.BlockSpec((1, 3 * _H), lambda i: (0, 0)),
        pl.BlockSpec((1, 3 * _H), lambda i: (0, 0)),
        pl.BlockSpec((_H, _H), lambda i: (0, 0)),
        pl.BlockSpec((1, _H), lambda i: (0, 0)),
    ]
    out_specs = pl.BlockSpec((_RB, _H), lambda i: (i, 0))
    out_shape = jax.ShapeDtypeStruct((_N, _H), jnp.float32)
    if final:
        out_specs = [out_specs, pl.BlockSpec((1, _H), lambda i: (0, 0))]
        out_shape = [out_shape, jax.ShapeDtypeStruct((1, _H), jnp.float32)]
    return pl.pallas_call(
        functools.partial(_tc_body, final),
        grid=(_N // _RB,),
        in_specs=in_specs,
        out_specs=out_specs,
        out_shape=out_shape,
    )


_tc_step_mid = _make_tc_step(False)
_tc_step_fin = _make_tc_step(True)


def kernel(initial_node_representation, adjacency_list_0, adjacency_list_1,
           W_msg_0_0, b_msg_0_0, W_msg_0_1, b_msg_0_1,
           W_ih_0, W_hh_0, b_ih_0, b_hh_0, fc_W, fc_b):
    def prep(adj):
        src = adj[:, 0].astype(jnp.int32)
        dst = adj[:, 1].astype(jnp.int32)
        src = jnp.concatenate([src, jnp.zeros((_EPAD - _E,), jnp.int32)])
        dst = jnp.concatenate([dst, jnp.full((_EPAD - _E,), _N, jnp.int32)])
        return src.reshape(_NS * _CPT, _CHUNK), dst.reshape(_NS * _CPT, _CHUNK)

    s0, d0 = prep(adjacency_list_0)
    s1, d1 = prep(adjacency_list_1)
    src = jnp.stack([s0, s1])
    dst = jnp.stack([d0, d1])

    zrows = jnp.zeros((_CHUNK, _H), jnp.float32)
    zdeg = jnp.zeros((_NP1,), jnp.float32)
    ones = jnp.ones((_CHUNK,), jnp.float32)

    wm = jnp.concatenate([W_msg_0_0.T, W_msg_0_1.T], axis=0)
    bm = jnp.zeros((8, _H), jnp.float32).at[0].set(b_msg_0_0).at[1].set(b_msg_0_1)
    wih = W_ih_0.T
    whh = W_hh_0.T
    bih = b_ih_0[None, :]
    bhh = b_hh_0[None, :]
    fcw = fc_W.T
    fcb = fc_b[None, :]

    h = initial_node_representation
    a, deg = _sc_agg(h, src, dst, zrows, zdeg, ones)
    deg8 = jnp.concatenate(
        [deg[0, :_N, None], deg[1, :_N, None],
         jnp.zeros((_N, 6), jnp.float32)], axis=1)
    h = _tc_step_mid(a, h, deg8, wm, bm, wih, whh, bih, bhh, fcw, fcb)
    a, _ = _sc_agg(h, src, dst, zrows, zdeg, ones)
    h, g = _tc_step_fin(a, h, deg8, wm, bm, wih, whh, bih, bhh, fcw, fcb)
    return g[0]
